# baseline (speedup 1.0000x reference)
import functools

import jax
import jax.numpy as jnp
from jax import lax
from jax.experimental import pallas as pl
from jax.experimental.pallas import tpu as pltpu
from jax.experimental.pallas import tpu_sc as plsc

BN_EPS = 1e-5
COS_EPS = 1e-8
LRELU_SLOPE = 0.01


def _p1_body(x_ref, w_ref, h_ref, s_ref):
    xb = x_ref[...]
    hb = lax.dot_general(xb, w_ref[...], (((1,), (1,)), ((), ())),
                         preferred_element_type=jnp.float32)
    h_ref[...] = hb
    s0 = jnp.sum(hb, axis=0, keepdims=True)
    s1 = jnp.sum(hb * hb, axis=0, keepdims=True)
    st = jnp.concatenate([s0, s1], axis=0)

    @pl.when(pl.program_id(0) == 0)
    def _():
        s_ref[...] = st

    @pl.when(pl.program_id(0) != 0)
    def _():
        s_ref[...] += st


def _encode_and_stats(x, w_conv):
    m, k = x.shape
    ls = w_conv.shape[0]
    bm = 2048
    return pl.pallas_call(
        _p1_body,
        grid=(m // bm,),
        in_specs=[
            pl.BlockSpec((bm, k), lambda i: (i, 0)),
            pl.BlockSpec((ls, k), lambda i: (0, 0)),
        ],
        out_specs=[
            pl.BlockSpec((bm, ls), lambda i: (i, 0)),
            pl.BlockSpec((2, ls), lambda i: (0, 0)),
        ],
        out_shape=[
            jax.ShapeDtypeStruct((m, ls), jnp.float32),
            jax.ShapeDtypeStruct((2, ls), jnp.float32),
        ],
        compiler_params=pltpu.CompilerParams(
            dimension_semantics=("arbitrary",)),
    )(x, w_conv)


def _sc_gather(h, idx3):
    info = plsc.get_sparse_core_info()
    nc, ns = info.num_cores, info.num_subcores
    nw = nc * ns
    m, ls = h.shape
    rpw = m // nw
    nch = idx3.shape[1]
    mesh = plsc.VectorSubcoreMesh(core_axis_name="c", subcore_axis_name="s")

    @functools.partial(
        pl.kernel,
        mesh=mesh,
        out_type=jax.ShapeDtypeStruct((m, ls), jnp.float32),
        scratch_types=[
            pltpu.VMEM((nch, 128), jnp.int32),
            pltpu.VMEM((rpw, ls), jnp.float32),
            pltpu.SemaphoreType.DMA,
        ],
        compiler_params=pltpu.CompilerParams(use_tc_tiling_on_sc=False),
    )
    def k(h_hbm, idx_hbm, out_hbm, idx_v, rows_v, sem):
        wid = lax.axis_index("s") * nc + lax.axis_index("c")
        base = wid * rpw
        pltpu.sync_copy(idx_hbm.at[wid], idx_v)
        copies = []
        for j in range(nch):
            copies.append(pltpu.async_copy(
                h_hbm.at[idx_v.at[j]], rows_v.at[pl.ds(j * 128, 128)], sem))
        for c in copies:
            c.wait()
        pltpu.sync_copy(rows_v, out_hbm.at[pl.ds(base, rpw)])

    return k(h, idx3)


def _p2_body(n_rows, h_ref, hp_ref, s_ref, g_ref, be_ref, w_ref, bl_ref,
             m_ref, o0_ref, o1_ref, l_ref):
    s = s_ref[...]
    mean = s[0:1, :] / n_rows
    var = s[1:2, :] / n_rows - mean * mean
    inv = lax.rsqrt(var + BN_EPS)
    scale = g_ref[...] * inv
    shift = be_ref[...] - mean * scale
    w = w_ref[...]
    bl = bl_ref[...]

    def head(hb):
        a = hb * scale + shift
        a = jnp.where(a >= 0, a, LRELU_SLOPE * a)
        return lax.dot_general(a, w, (((1,), (1,)), ((), ())),
                               preferred_element_type=jnp.float32) + bl

    z = head(h_ref[0])
    zp = head(hp_ref[0])
    zn = jnp.concatenate([z[1:], z[:1]], axis=0)

    na = jnp.sqrt(jnp.sum(z * z, axis=-1, keepdims=True))
    nn = jnp.concatenate([na[1:], na[:1]], axis=0)
    np_ = jnp.sqrt(jnp.sum(zp * zp, axis=-1, keepdims=True))

    na_c = jnp.maximum(na, COS_EPS)
    pos = jnp.sum(z * zn, axis=-1, keepdims=True) / (na_c * jnp.maximum(nn, COS_EPS))
    neg = jnp.sum(z * zp, axis=-1, keepdims=True) / (na_c * jnp.maximum(np_, COS_EPS))

    mx = jnp.maximum(pos, neg)
    lse = mx + jnp.log(jnp.exp(pos - mx) + jnp.exp(neg - mx))
    o0 = pos - lse
    o1 = neg - lse
    o0_ref[...] = o0[None]
    o1_ref[...] = o1[None]
    l_ref[...] = (-o0 * (1.0 - m_ref[0]))[None]


def _score(h3, hp3, stats, gamma2, beta2, w_lin, b_lin2, mask3):
    b, t, ls = h3.shape
    n_rows = b * t
    full = lambda bi: (bi, 0, 0)
    const2 = lambda bi: (0, 0)
    out_spec = pl.BlockSpec((1, t, 1), full)
    return pl.pallas_call(
        functools.partial(_p2_body, float(n_rows)),
        grid=(b,),
        in_specs=[
            pl.BlockSpec((1, t, ls), full),
            pl.BlockSpec((1, t, ls), full),
            pl.BlockSpec((2, ls), const2),
            pl.BlockSpec((1, ls), const2),
            pl.BlockSpec((1, ls), const2),
            pl.BlockSpec((ls, ls), const2),
            pl.BlockSpec((1, ls), const2),
            pl.BlockSpec((1, t, 1), full),
        ],
        out_specs=[out_spec, out_spec, out_spec],
        out_shape=[jax.ShapeDtypeStruct((b, t, 1), jnp.float32)] * 3,
        compiler_params=pltpu.CompilerParams(
            dimension_semantics=("parallel",)),
    )(h3, hp3, stats, gamma2, beta2, w_lin, b_lin2, mask3)


def kernel(logits, padding_mask, W_conv, gamma, beta, W_lin, b_lin):
    b, t, i_dim = logits.shape
    ls = W_conv.shape[0]

    x = logits.reshape(b * t, i_dim)
    h, stats = _encode_and_stats(x, W_conv)

    perm = jax.random.permutation(
        jax.random.fold_in(jax.random.key(42), 0), t - 1)
    perm_full = jnp.concatenate(
        [perm.astype(jnp.int32), jnp.array([t - 1], jnp.int32)])
    idx = (jnp.arange(b, dtype=jnp.int32)[:, None] * t
           + perm_full[None, :]).reshape(-1)
    idx3 = idx.reshape(32, -1, 128)

    hp = _sc_gather(h, idx3)

    out0, out1, loss = _score(
        h.reshape(b, t, ls), hp.reshape(b, t, ls), stats,
        gamma.reshape(1, ls), beta.reshape(1, ls), W_lin,
        b_lin.reshape(1, ls),
        padding_mask.astype(jnp.float32).reshape(b, t, 1))

    out = jnp.stack(
        [out0.reshape(b, t)[:, :t - 1], out1.reshape(b, t)[:, :t - 1]],
        axis=-1)
    return (out, loss.reshape(b, t)[:, :t - 1])


# speedup vs baseline: 1.8688x; 1.8688x over previous
import functools

import jax
import jax.numpy as jnp
from jax import lax
from jax.experimental import pallas as pl
from jax.experimental.pallas import tpu as pltpu
from jax.experimental.pallas import tpu_sc as plsc

BN_EPS = 1e-5
COS_EPS = 1e-8
LRELU_SLOPE = 0.01


def _p1_body(x_ref, w_ref, h_ref, ht_ref, st_ref, sr_ref):
    xb = x_ref[...]
    w = w_ref[...]
    hb = lax.dot_general(xb, w, (((1,), (1,)), ((), ())),
                         preferred_element_type=jnp.float32)
    hbt = lax.dot_general(w, xb, (((1,), (1,)), ((), ())),
                          preferred_element_type=jnp.float32)
    h_ref[...] = hb
    ht_ref[...] = hbt
    st = jnp.concatenate(
        [jnp.sum(hbt, axis=1, keepdims=True),
         jnp.sum(hbt * hbt, axis=1, keepdims=True)], axis=1)
    sr = jnp.concatenate(
        [jnp.sum(hb, axis=0, keepdims=True),
         jnp.sum(hb * hb, axis=0, keepdims=True)], axis=0)

    @pl.when(pl.program_id(0) == 0)
    def _():
        st_ref[...] = st
        sr_ref[...] = sr

    @pl.when(pl.program_id(0) != 0)
    def _():
        st_ref[...] += st
        sr_ref[...] += sr


def _encode_and_stats(x, w_conv):
    m, k = x.shape
    ls = w_conv.shape[0]
    bm = 2048
    return pl.pallas_call(
        _p1_body,
        grid=(m // bm,),
        in_specs=[
            pl.BlockSpec((bm, k), lambda i: (i, 0)),
            pl.BlockSpec((ls, k), lambda i: (0, 0)),
        ],
        out_specs=[
            pl.BlockSpec((bm, ls), lambda i: (i, 0)),
            pl.BlockSpec((ls, bm), lambda i: (0, i)),
            pl.BlockSpec((ls, 2), lambda i: (0, 0)),
            pl.BlockSpec((2, ls), lambda i: (0, 0)),
        ],
        out_shape=[
            jax.ShapeDtypeStruct((m, ls), jnp.float32),
            jax.ShapeDtypeStruct((ls, m), jnp.float32),
            jax.ShapeDtypeStruct((ls, 2), jnp.float32),
            jax.ShapeDtypeStruct((2, ls), jnp.float32),
        ],
        compiler_params=pltpu.CompilerParams(
            dimension_semantics=("arbitrary",)),
    )(x, w_conv)


def _sc_gather(h, idx3):
    info = plsc.get_sparse_core_info()
    nc, ns = info.num_cores, info.num_subcores
    nw = nc * ns
    m, ls = h.shape
    rpw = m // nw
    nch = idx3.shape[1]
    mesh = plsc.VectorSubcoreMesh(core_axis_name="c", subcore_axis_name="s")

    @functools.partial(
        pl.kernel,
        mesh=mesh,
        out_type=jax.ShapeDtypeStruct((m, ls), jnp.float32),
        scratch_types=[
            pltpu.VMEM((nch, 128), jnp.int32),
            pltpu.VMEM((rpw, ls), jnp.float32),
            pltpu.SemaphoreType.DMA,
        ],
        compiler_params=pltpu.CompilerParams(use_tc_tiling_on_sc=False),
    )
    def k(h_hbm, idx_hbm, out_hbm, idx_v, rows_v, sem):
        wid = lax.axis_index("s") * nc + lax.axis_index("c")
        base = wid * rpw
        pltpu.sync_copy(idx_hbm.at[wid], idx_v)
        copies = []
        for j in range(nch):
            copies.append(pltpu.async_copy(
                h_hbm.at[idx_v.at[j]], rows_v.at[pl.ds(j * 128, 128)], sem))
        for c in copies:
            c.wait()
        pltpu.sync_copy(rows_v, out_hbm.at[pl.ds(base, rpw)])

    return k(h, idx3)


def _lane_roll(x):
    return jnp.concatenate([x[:, 1:], x[:, :1]], axis=1)


def _p2_body(n_rows, ht_ref, hp_ref, st_ref, sr_ref, gt_ref, bt_ref,
             gr_ref, br_ref, w_ref, blt_ref, m_ref, o0_ref, o1_ref, l_ref):
    w = w_ref[...]

    st = st_ref[...]
    mean_t = st[:, 0:1] / n_rows
    var_t = st[:, 1:2] / n_rows - mean_t * mean_t
    scale_t = gt_ref[...] * lax.rsqrt(var_t + BN_EPS)
    shift_t = bt_ref[...] - mean_t * scale_t
    a_t = ht_ref[...] * scale_t + shift_t
    a_t = jnp.where(a_t >= 0, a_t, LRELU_SLOPE * a_t)
    z = lax.dot_general(w, a_t, (((1,), (0,)), ((), ())),
                        preferred_element_type=jnp.float32) + blt_ref[...]

    sr = sr_ref[...]
    mean_r = sr[0:1, :] / n_rows
    var_r = sr[1:2, :] / n_rows - mean_r * mean_r
    scale_r = gr_ref[...] * lax.rsqrt(var_r + BN_EPS)
    shift_r = br_ref[...] - mean_r * scale_r
    ap = hp_ref[0] * scale_r + shift_r
    ap = jnp.where(ap >= 0, ap, LRELU_SLOPE * ap)
    zp = lax.dot_general(w, ap, (((1,), (1,)), ((), ())),
                         preferred_element_type=jnp.float32) + blt_ref[...]

    zn = _lane_roll(z)
    r = 1.0 / jnp.maximum(
        jnp.sqrt(jnp.sum(z * z, axis=0, keepdims=True)), COS_EPS)
    rp = 1.0 / jnp.maximum(
        jnp.sqrt(jnp.sum(zp * zp, axis=0, keepdims=True)), COS_EPS)
    rn = _lane_roll(r)

    pos = jnp.sum(z * zn, axis=0, keepdims=True) * (r * rn)
    neg = jnp.sum(z * zp, axis=0, keepdims=True) * (r * rp)

    mx = jnp.maximum(pos, neg)
    lse = mx + jnp.log(jnp.exp(pos - mx) + jnp.exp(neg - mx))
    o0 = pos - lse
    o0_ref[...] = o0[None]
    o1_ref[...] = (neg - lse)[None]
    l_ref[...] = (-o0 * (1.0 - m_ref[0]))[None]


def _score(ht, hp3, st, sr, gamma, beta, w_lin, b_lin, mask3, b, t):
    ls = ht.shape[0]
    n_rows = float(b * t)
    out_spec = pl.BlockSpec((1, 1, t), lambda bi: (bi, 0, 0))
    return pl.pallas_call(
        functools.partial(_p2_body, n_rows),
        grid=(b,),
        in_specs=[
            pl.BlockSpec((ls, t), lambda bi: (0, bi)),
            pl.BlockSpec((1, t, ls), lambda bi: (bi, 0, 0)),
            pl.BlockSpec((ls, 2), lambda bi: (0, 0)),
            pl.BlockSpec((2, ls), lambda bi: (0, 0)),
            pl.BlockSpec((ls, 1), lambda bi: (0, 0)),
            pl.BlockSpec((ls, 1), lambda bi: (0, 0)),
            pl.BlockSpec((1, ls), lambda bi: (0, 0)),
            pl.BlockSpec((1, ls), lambda bi: (0, 0)),
            pl.BlockSpec((ls, ls), lambda bi: (0, 0)),
            pl.BlockSpec((ls, 1), lambda bi: (0, 0)),
            pl.BlockSpec((1, 1, t), lambda bi: (bi, 0, 0)),
        ],
        out_specs=[out_spec, out_spec, out_spec],
        out_shape=[jax.ShapeDtypeStruct((b, 1, t), jnp.float32)] * 3,
        compiler_params=pltpu.CompilerParams(
            dimension_semantics=("parallel",)),
    )(ht, hp3, st, sr, gamma.reshape(ls, 1), beta.reshape(ls, 1),
      gamma.reshape(1, ls), beta.reshape(1, ls), w_lin,
      b_lin.reshape(ls, 1), mask3)


def kernel(logits, padding_mask, W_conv, gamma, beta, W_lin, b_lin):
    b, t, i_dim = logits.shape
    ls = W_conv.shape[0]

    x = logits.reshape(b * t, i_dim)
    h, ht, st, sr = _encode_and_stats(x, W_conv)

    with jax.ensure_compile_time_eval():
        perm = jax.random.permutation(
            jax.random.fold_in(jax.random.key(42), 0), t - 1)
        perm_full = jnp.concatenate(
            [perm.astype(jnp.int32), jnp.array([t - 1], jnp.int32)])
        idx = (jnp.arange(b, dtype=jnp.int32)[:, None] * t
               + perm_full[None, :]).reshape(-1)
        idx3 = idx.reshape(32, -1, 128)

    hp = _sc_gather(h, idx3)

    out0, out1, loss = _score(
        ht, hp.reshape(b, t, ls), st, sr, gamma, beta, W_lin, b_lin,
        padding_mask.astype(jnp.float32).reshape(b, 1, t), b, t)

    out = jnp.stack(
        [out0.reshape(b, t)[:, :t - 1], out1.reshape(b, t)[:, :t - 1]],
        axis=-1)
    return (out, loss.reshape(b, t)[:, :t - 1])
